# baseline (speedup 1.0000x reference)
import jax
import jax.numpy as jnp
from jax import lax
from jax.experimental import pallas as pl
from jax.experimental.pallas import tpu as pltpu

N = 2048
C = 1024
E = 8
BN = 1024
NB = N // BN

_F32 = jnp.float32


def _moe_block_kernel(
    x_ref, de_ref, ws_ref, bs_ref, wc_ref, bc_ref, g_ref, bt_ref,
    wg_ref, bg_ref, we_ref, be_ref,
    y_ref, loss_ref,
    wte_ref, sums_ref,
):
    i = pl.program_id(0)
    e = pl.program_id(1)
    x = x_ref[...]
    x_bf = x.astype(jnp.bfloat16)

    @pl.when(e == 0)
    def _gating_and_share():
        conv = lax.dot_general(
            x, wc_ref[...], (((1,), (1,)), ((), ())),
            preferred_element_type=_F32) + bc_ref[...]
        mu = jnp.mean(conv, axis=1, keepdims=True)
        var = jnp.mean((conv - mu) ** 2, axis=1, keepdims=True)
        route = ((conv - mu) * lax.rsqrt(var + 1e-5) * g_ref[...]
                 + bt_ref[...] + de_ref[...])
        logits = lax.dot_general(
            route, wg_ref[...], (((1,), (1,)), ((), ())),
            preferred_element_type=_F32) + bg_ref[...]
        m = jnp.max(logits, axis=1, keepdims=True)
        ex = jnp.exp(logits - m)
        w = ex / jnp.sum(ex, axis=1, keepdims=True)
        ent = -jnp.sum(w * jnp.log(w + 1e-12), axis=1, keepdims=True)
        kf = jnp.clip(jnp.ceil(1.0 + (ent / jnp.log(8.0)) * 7.0), 1.0, 8.0)
        cols = [w[:, j:j + 1] for j in range(E)]
        sel = []
        for ei in range(E):
            r = jnp.zeros((BN, 1), _F32)
            for j in range(E):
                if j == ei:
                    continue
                beats = cols[j] > cols[ei]
                if j < ei:
                    beats = beats | (cols[j] == cols[ei])
                r += beats.astype(_F32)
            sel.append(r < kf)
        wte_ref[...] = jnp.concatenate(
            [jnp.where(sel[ei], cols[ei], 0.0) for ei in range(E)], axis=1)
        mask_sums = jnp.concatenate(
            [jnp.sum(sel[ei].astype(_F32), axis=0, keepdims=True)
             for ei in range(E)], axis=1)
        w_sums = jnp.sum(w, axis=0, keepdims=True)
        part = jnp.concatenate([mask_sums, w_sums], axis=0)

        @pl.when(i == 0)
        def _():
            sums_ref[...] = part

        @pl.when(i != 0)
        def _():
            sums_ref[...] += part

        y_ref[...] = lax.dot_general(
            x_bf, ws_ref[...], (((1,), (1,)), ((), ())),
            preferred_element_type=_F32) + bs_ref[...]

    xw = lax.dot_general(
        x_bf, we_ref[0], (((1,), (1,)), ((), ())),
        preferred_element_type=_F32) + be_ref[0]
    onehot = (lax.broadcasted_iota(jnp.int32, (E, 1), 0) == e).astype(_F32)
    wcol = lax.dot_general(
        wte_ref[...], onehot, (((1,), (0,)), ((), ())),
        preferred_element_type=_F32)
    y_ref[...] += wcol * xw

    @pl.when((i == NB - 1) & (e == E - 1))
    def _finalize_loss():
        s = sums_ref[...]
        tpe = s[0:1, :] * (1.0 / N)
        rpp = s[1:2, :] * (1.0 / N)
        loss_ref[...] = jnp.sum(tpe * rpp, axis=1, keepdims=True) * (
            float(E * E) / float(E))


@jax.jit
def _moe(features, domain_emb, W_share, b_share, W_conv, b_conv,
         ln_gamma, ln_beta, W_gate, b_gate, W_experts, b_experts):
    de = domain_emb.reshape(1, C)
    bs = b_share.reshape(1, C)
    bc = b_conv.reshape(1, C)
    g = ln_gamma.reshape(1, C)
    bt = ln_beta.reshape(1, C)
    bg = b_gate.reshape(1, E)
    be = b_experts.reshape(E, 1, C)
    ws_bf = W_share.astype(jnp.bfloat16)
    we_bf = W_experts.astype(jnp.bfloat16)

    full = lambda *_: (0, 0)
    grid = (NB, E)
    y, loss = pl.pallas_call(
        _moe_block_kernel,
        grid=grid,
        in_specs=[
            pl.BlockSpec((BN, C), lambda i, e: (i, 0)),
            pl.BlockSpec((1, C), full),
            pl.BlockSpec((C, C), full),
            pl.BlockSpec((1, C), full),
            pl.BlockSpec((C, C), full),
            pl.BlockSpec((1, C), full),
            pl.BlockSpec((1, C), full),
            pl.BlockSpec((1, C), full),
            pl.BlockSpec((E, C), full),
            pl.BlockSpec((1, E), full),
            pl.BlockSpec((1, C, C), lambda i, e: (e, 0, 0)),
            pl.BlockSpec((1, 1, C), lambda i, e: (e, 0, 0)),
        ],
        out_specs=[
            pl.BlockSpec((BN, C), lambda i, e: (i, 0)),
            pl.BlockSpec((1, 1), full),
        ],
        out_shape=[
            jax.ShapeDtypeStruct((N, C), _F32),
            jax.ShapeDtypeStruct((1, 1), _F32),
        ],
        scratch_shapes=[
            pltpu.VMEM((BN, E), _F32),
            pltpu.VMEM((2, E), _F32),
        ],
        compiler_params=pltpu.CompilerParams(
            dimension_semantics=("arbitrary", "arbitrary"),
        ),
    )(features, de, ws_bf, bs, W_conv, bc, g, bt, W_gate, bg,
      we_bf, be)
    return y, loss[0, 0]


def kernel(features, domain_emb, W_share, b_share, W_conv, b_conv,
           ln_gamma, ln_beta, W_gate, b_gate, W_experts, b_experts):
    return _moe(features, domain_emb, W_share, b_share, W_conv, b_conv,
                ln_gamma, ln_beta, W_gate, b_gate, W_experts, b_experts)


# speedup vs baseline: 1.2337x; 1.2337x over previous
import jax
import jax.numpy as jnp
from jax import lax
from jax.experimental import pallas as pl
from jax.experimental.pallas import tpu as pltpu

N = 2048
C = 1024
E = 8
BN = 1024
NB = N // BN

_F32 = jnp.float32


def _moe_block_kernel(
    x_ref, de_ref, ws_ref, bs_ref, wc_ref, bc_ref, g_ref, bt_ref,
    wg_ref, bg_ref, we_ref, be_ref,
    y_ref, loss_ref,
    wte_ref, sums_ref,
):
    i = pl.program_id(0)
    e = pl.program_id(1)
    x = x_ref[...]

    @pl.when(e == 0)
    def _gating_and_share():
        conv = lax.dot_general(
            x, wc_ref[...], (((1,), (1,)), ((), ())),
            preferred_element_type=_F32) + bc_ref[...]
        mu = jnp.mean(conv, axis=1, keepdims=True)
        var = jnp.mean((conv - mu) ** 2, axis=1, keepdims=True)
        route = ((conv - mu) * lax.rsqrt(var + 1e-5) * g_ref[...]
                 + bt_ref[...] + de_ref[...])
        logits = lax.dot_general(
            route, wg_ref[...], (((1,), (1,)), ((), ())),
            preferred_element_type=_F32) + bg_ref[...]
        m = jnp.max(logits, axis=1, keepdims=True)
        ex = jnp.exp(logits - m)
        w = ex / jnp.sum(ex, axis=1, keepdims=True)
        ent = -jnp.sum(w * jnp.log(w + 1e-12), axis=1, keepdims=True)
        kf = jnp.clip(jnp.ceil(1.0 + (ent / jnp.log(8.0)) * 7.0), 1.0, 8.0)
        cols = [w[:, j:j + 1] for j in range(E)]
        sel = []
        for ei in range(E):
            r = jnp.zeros((BN, 1), _F32)
            for j in range(E):
                if j == ei:
                    continue
                beats = cols[j] > cols[ei]
                if j < ei:
                    beats = beats | (cols[j] == cols[ei])
                r += beats.astype(_F32)
            sel.append(r < kf)
        wte_ref[...] = jnp.concatenate(
            [jnp.where(sel[ei], cols[ei], 0.0) for ei in range(E)], axis=1)
        mask_sums = jnp.concatenate(
            [jnp.sum(sel[ei].astype(_F32), axis=0, keepdims=True)
             for ei in range(E)], axis=1)
        w_sums = jnp.sum(w, axis=0, keepdims=True)
        part = jnp.concatenate([mask_sums, w_sums], axis=0)

        @pl.when(i == 0)
        def _():
            sums_ref[...] = part

        @pl.when(i != 0)
        def _():
            sums_ref[...] += part

        y_ref[...] = lax.dot_general(
            x, ws_ref[...], (((1,), (1,)), ((), ())),
            preferred_element_type=_F32) + bs_ref[...]

    xw = lax.dot_general(
        x, we_ref[0], (((1,), (1,)), ((), ())),
        preferred_element_type=_F32) + be_ref[0]
    onehot = (lax.broadcasted_iota(jnp.int32, (E, 1), 0) == e).astype(_F32)
    wcol = lax.dot_general(
        wte_ref[...], onehot, (((1,), (0,)), ((), ())),
        preferred_element_type=_F32)
    y_ref[...] += wcol * xw

    @pl.when((i == NB - 1) & (e == E - 1))
    def _finalize_loss():
        s = sums_ref[...]
        tpe = s[0:1, :] * (1.0 / N)
        rpp = s[1:2, :] * (1.0 / N)
        loss_ref[...] = jnp.sum(tpe * rpp, axis=1, keepdims=True) * (
            float(E * E) / float(E))


@jax.jit
def _moe(features, domain_emb, W_share, b_share, W_conv, b_conv,
         ln_gamma, ln_beta, W_gate, b_gate, W_experts, b_experts):
    de = domain_emb.reshape(1, C)
    bs = b_share.reshape(1, C)
    bc = b_conv.reshape(1, C)
    g = ln_gamma.reshape(1, C)
    bt = ln_beta.reshape(1, C)
    bg = b_gate.reshape(1, E)
    be = b_experts.reshape(E, 1, C)

    full = lambda *_: (0, 0)
    grid = (NB, E)
    y, loss = pl.pallas_call(
        _moe_block_kernel,
        grid=grid,
        in_specs=[
            pl.BlockSpec((BN, C), lambda i, e: (i, 0)),
            pl.BlockSpec((1, C), full),
            pl.BlockSpec((C, C), full),
            pl.BlockSpec((1, C), full),
            pl.BlockSpec((C, C), full),
            pl.BlockSpec((1, C), full),
            pl.BlockSpec((1, C), full),
            pl.BlockSpec((1, C), full),
            pl.BlockSpec((E, C), full),
            pl.BlockSpec((1, E), full),
            pl.BlockSpec((1, C, C), lambda i, e: (e, 0, 0)),
            pl.BlockSpec((1, 1, C), lambda i, e: (e, 0, 0)),
        ],
        out_specs=[
            pl.BlockSpec((BN, C), lambda i, e: (i, 0)),
            pl.BlockSpec((1, 1), full),
        ],
        out_shape=[
            jax.ShapeDtypeStruct((N, C), _F32),
            jax.ShapeDtypeStruct((1, 1), _F32),
        ],
        scratch_shapes=[
            pltpu.VMEM((BN, E), _F32),
            pltpu.VMEM((2, E), _F32),
        ],
        compiler_params=pltpu.CompilerParams(
            dimension_semantics=("arbitrary", "arbitrary"),
        ),
    )(features, de, W_share, bs, W_conv, bc, g, bt, W_gate, bg,
      W_experts, be)
    return y, loss[0, 0]


def kernel(features, domain_emb, W_share, b_share, W_conv, b_conv,
           ln_gamma, ln_beta, W_gate, b_gate, W_experts, b_experts):
    return _moe(features, domain_emb, W_share, b_share, W_conv, b_conv,
                ln_gamma, ln_beta, W_gate, b_gate, W_experts, b_experts)


# speedup vs baseline: 1.5725x; 1.2746x over previous
import jax
import jax.numpy as jnp
from jax import lax
from jax.experimental import pallas as pl
from jax.experimental.pallas import tpu as pltpu

N = 2048
C = 1024
E = 8
BN = 2048
NB = N // BN

_F32 = jnp.float32


def _moe_block_kernel(
    x_ref, de_ref, ws_ref, bs_ref, wc_ref, bc_ref, g_ref, bt_ref,
    wg_ref, bg_ref, we_ref, be_ref,
    y_ref, loss_ref,
    wte_ref, sums_ref,
):
    i = pl.program_id(0)
    e = pl.program_id(1)
    x = x_ref[...]

    @pl.when(e == 0)
    def _gating_and_share():
        conv = lax.dot_general(
            x, wc_ref[...], (((1,), (1,)), ((), ())),
            preferred_element_type=_F32) + bc_ref[...]
        mu = jnp.mean(conv, axis=1, keepdims=True)
        var = jnp.mean((conv - mu) ** 2, axis=1, keepdims=True)
        route = ((conv - mu) * lax.rsqrt(var + 1e-5) * g_ref[...]
                 + bt_ref[...] + de_ref[...])
        logits = lax.dot_general(
            wg_ref[...], route, (((1,), (1,)), ((), ())),
            preferred_element_type=_F32) + bg_ref[...]
        m = jnp.max(logits, axis=0, keepdims=True)
        ex = jnp.exp(logits - m)
        w = ex / jnp.sum(ex, axis=0, keepdims=True)
        ent = -jnp.sum(w * jnp.log(w + 1e-12), axis=0, keepdims=True)
        kf = jnp.clip(jnp.ceil(1.0 + (ent / jnp.log(8.0)) * 7.0), 1.0, 8.0)
        row = lax.broadcasted_iota(jnp.int32, (E, 1), 0)
        rank = jnp.zeros_like(w)
        for d in range(1, E):
            wj = pltpu.roll(w, E - d, 0)
            tie_break = ((row + d) % E) < row
            beats = (wj > w) | ((wj == w) & tie_break)
            rank += beats.astype(_F32)
        sel = rank < kf
        wte_t = jnp.where(sel, w, 0.0)
        wte_ref[...] = wte_t.T
        mask_sums = jnp.sum(sel.astype(_F32), axis=1, keepdims=True)
        w_sums = jnp.sum(w, axis=1, keepdims=True)
        part = jnp.concatenate([mask_sums, w_sums], axis=1)

        @pl.when(i == 0)
        def _():
            sums_ref[...] = part

        @pl.when(i != 0)
        def _():
            sums_ref[...] += part

        y_ref[...] = lax.dot_general(
            x, ws_ref[...], (((1,), (1,)), ((), ())),
            preferred_element_type=_F32) + bs_ref[...]

    xw = lax.dot_general(
        x, we_ref[0], (((1,), (1,)), ((), ())),
        preferred_element_type=_F32) + be_ref[0]
    onehot = (lax.broadcasted_iota(jnp.int32, (E, 1), 0) == e).astype(_F32)
    wcol = lax.dot_general(
        wte_ref[...], onehot, (((1,), (0,)), ((), ())),
        preferred_element_type=_F32)
    y_ref[...] += wcol * xw

    @pl.when((i == NB - 1) & (e == E - 1))
    def _finalize_loss():
        s = sums_ref[...]
        prod = s[:, 0:1] * s[:, 1:2] * (1.0 / (N * N))
        loss_ref[...] = jnp.sum(prod, axis=0, keepdims=True) * (
            float(E * E) / float(E))


@jax.jit
def _moe(features, domain_emb, W_share, b_share, W_conv, b_conv,
         ln_gamma, ln_beta, W_gate, b_gate, W_experts, b_experts):
    de = domain_emb.reshape(1, C)
    bs = b_share.reshape(1, C)
    bc = b_conv.reshape(1, C)
    g = ln_gamma.reshape(1, C)
    bt = ln_beta.reshape(1, C)
    bg = b_gate.reshape(E, 1)
    be = b_experts.reshape(E, 1, C)

    full = lambda *_: (0, 0)
    grid = (NB, E)
    y, loss = pl.pallas_call(
        _moe_block_kernel,
        grid=grid,
        in_specs=[
            pl.BlockSpec((BN, C), lambda i, e: (i, 0)),
            pl.BlockSpec((1, C), full),
            pl.BlockSpec((C, C), full),
            pl.BlockSpec((1, C), full),
            pl.BlockSpec((C, C), full),
            pl.BlockSpec((1, C), full),
            pl.BlockSpec((1, C), full),
            pl.BlockSpec((1, C), full),
            pl.BlockSpec((E, C), full),
            pl.BlockSpec((E, 1), full),
            pl.BlockSpec((1, C, C), lambda i, e: (e, 0, 0)),
            pl.BlockSpec((1, 1, C), lambda i, e: (e, 0, 0)),
        ],
        out_specs=[
            pl.BlockSpec((BN, C), lambda i, e: (i, 0)),
            pl.BlockSpec((1, 1), full),
        ],
        out_shape=[
            jax.ShapeDtypeStruct((N, C), _F32),
            jax.ShapeDtypeStruct((1, 1), _F32),
        ],
        scratch_shapes=[
            pltpu.VMEM((BN, E), _F32),
            pltpu.VMEM((E, 2), _F32),
        ],
        compiler_params=pltpu.CompilerParams(
            dimension_semantics=("arbitrary", "arbitrary"),
        ),
    )(features, de, W_share, bs, W_conv, bc, g, bt, W_gate, bg,
      W_experts, be)
    return y, loss[0, 0]


def kernel(features, domain_emb, W_share, b_share, W_conv, b_conv,
           ln_gamma, ln_beta, W_gate, b_gate, W_experts, b_experts):
    return _moe(features, domain_emb, W_share, b_share, W_conv, b_conv,
                ln_gamma, ln_beta, W_gate, b_gate, W_experts, b_experts)
